# baseline (speedup 1.0000x reference)
import jax
import jax.numpy as jnp
from jax.experimental import pallas as pl

TILE_M = 1024


def _mlp_body(x_ref, w0_ref, b0_ref, w1_ref, b1_ref, w2_ref, b2_ref, out_ref):
    xb = x_ref[...].astype(jnp.bfloat16)
    w0 = w0_ref[...].astype(jnp.bfloat16)
    h = jnp.dot(xb, w0, preferred_element_type=jnp.float32)
    h = jnp.maximum(h + b0_ref[...], 0.0).astype(jnp.bfloat16)
    w1 = w1_ref[...].astype(jnp.bfloat16)
    h = jnp.dot(h, w1, preferred_element_type=jnp.float32)
    h = jnp.maximum(h + b1_ref[...], 0.0).astype(jnp.bfloat16)
    w2 = w2_ref[...].astype(jnp.bfloat16)
    z = jnp.dot(h, w2, preferred_element_type=jnp.float32)
    out_ref[...] = z + b2_ref[...]


def _softmax_body(z_ref, o_ref):
    z = z_ref[...]
    m = jnp.max(z, axis=1, keepdims=True)
    e = jnp.exp(z - m)
    o_ref[...] = e / jnp.sum(e, axis=1, keepdims=True)


def kernel(x, W0, b0, W1, b1, W2, b2):
    B, S, D = x.shape
    H0 = W0.shape[1]
    H1 = W1.shape[1]
    E = W2.shape[1]
    M = B * S
    xf = x.reshape(M, D)

    logits = pl.pallas_call(
        _mlp_body,
        grid=(M // TILE_M,),
        in_specs=[
            pl.BlockSpec((TILE_M, D), lambda i: (i, 0)),
            pl.BlockSpec((D, H0), lambda i: (0, 0)),
            pl.BlockSpec((1, H0), lambda i: (0, 0)),
            pl.BlockSpec((H0, H1), lambda i: (0, 0)),
            pl.BlockSpec((1, H1), lambda i: (0, 0)),
            pl.BlockSpec((H1, E), lambda i: (0, 0)),
            pl.BlockSpec((1, E), lambda i: (0, 0)),
        ],
        out_specs=pl.BlockSpec((TILE_M, E), lambda i: (i, 0)),
        out_shape=jax.ShapeDtypeStruct((M, E), jnp.float32),
    )(xf, W0, b0.reshape(1, H0), W1, b1.reshape(1, H1), W2, b2.reshape(1, E))

    logits = logits.reshape(B, S, E)
    return pl.pallas_call(
        _softmax_body,
        grid=(B,),
        in_specs=[pl.BlockSpec((1, S, E), lambda b: (b, 0, 0))],
        out_specs=pl.BlockSpec((1, S, E), lambda b: (b, 0, 0)),
        out_shape=jax.ShapeDtypeStruct((B, S, E), jnp.float32),
    )(logits)


# speedup vs baseline: 1.0010x; 1.0010x over previous
import jax
import jax.numpy as jnp
from jax.experimental import pallas as pl

TILE_M = 2048


def _mlp_body(x_ref, w0_ref, b0_ref, w1_ref, b1_ref, w2_ref, b2_ref, out_ref):
    h = jnp.dot(x_ref[...], w0_ref[...], preferred_element_type=jnp.float32)
    h = jnp.maximum(h + b0_ref[...], 0.0)
    h = jnp.dot(h, w1_ref[...], preferred_element_type=jnp.float32)
    h = jnp.maximum(h + b1_ref[...], 0.0)
    z = jnp.dot(h, w2_ref[...], preferred_element_type=jnp.float32)
    out_ref[...] = z + b2_ref[...]


def _softmax_body(z_ref, o_ref):
    z = z_ref[...]
    m = jnp.max(z, axis=1, keepdims=True)
    e = jnp.exp(z - m)
    o_ref[...] = e / jnp.sum(e, axis=1, keepdims=True)


def kernel(x, W0, b0, W1, b1, W2, b2):
    B, S, D = x.shape
    H0 = W0.shape[1]
    H1 = W1.shape[1]
    E = W2.shape[1]
    M = B * S
    xf = x.reshape(M, D)

    logits = pl.pallas_call(
        _mlp_body,
        grid=(M // TILE_M,),
        in_specs=[
            pl.BlockSpec((TILE_M, D), lambda i: (i, 0)),
            pl.BlockSpec((D, H0), lambda i: (0, 0)),
            pl.BlockSpec((1, H0), lambda i: (0, 0)),
            pl.BlockSpec((H0, H1), lambda i: (0, 0)),
            pl.BlockSpec((1, H1), lambda i: (0, 0)),
            pl.BlockSpec((H1, E), lambda i: (0, 0)),
            pl.BlockSpec((1, E), lambda i: (0, 0)),
        ],
        out_specs=pl.BlockSpec((TILE_M, E), lambda i: (i, 0)),
        out_shape=jax.ShapeDtypeStruct((M, E), jnp.float32),
    )(xf, W0, b0.reshape(1, H0), W1, b1.reshape(1, H1), W2, b2.reshape(1, E))

    logits = logits.reshape(B, S, E)
    return pl.pallas_call(
        _softmax_body,
        grid=(B,),
        in_specs=[pl.BlockSpec((1, S, E), lambda b: (b, 0, 0))],
        out_specs=pl.BlockSpec((1, S, E), lambda b: (b, 0, 0)),
        out_shape=jax.ShapeDtypeStruct((B, S, E), jnp.float32),
    )(logits)


# speedup vs baseline: 1.0028x; 1.0017x over previous
import jax
import jax.numpy as jnp
from jax.experimental import pallas as pl
from jax.experimental.pallas import tpu as pltpu

TILE_M = 2048


def _mlp_body(x_ref, w0_ref, b0_ref, w1_ref, b1_ref, w2_ref, b2_ref, out_ref):
    h = jnp.dot(x_ref[...], w0_ref[...], preferred_element_type=jnp.float32)
    h = jnp.maximum(h + b0_ref[...], 0.0)
    h = jnp.dot(h, w1_ref[...], preferred_element_type=jnp.float32)
    h = jnp.maximum(h + b1_ref[...], 0.0)
    z = jnp.dot(h, w2_ref[...], preferred_element_type=jnp.float32)
    out_ref[...] = z + b2_ref[...]


def _softmax_body(z_ref, o_ref):
    z = z_ref[...]
    m = jnp.max(z, axis=1, keepdims=True)
    e = jnp.exp(z - m)
    o_ref[...] = e / jnp.sum(e, axis=1, keepdims=True)


def kernel(x, W0, b0, W1, b1, W2, b2):
    B, S, D = x.shape
    H0 = W0.shape[1]
    H1 = W1.shape[1]
    E = W2.shape[1]
    M = B * S
    xf = x.reshape(M, D)

    logits = pl.pallas_call(
        _mlp_body,
        grid=(M // TILE_M,),
        in_specs=[
            pl.BlockSpec((TILE_M, D), lambda i: (i, 0)),
            pl.BlockSpec((D, H0), lambda i: (0, 0)),
            pl.BlockSpec((1, H0), lambda i: (0, 0)),
            pl.BlockSpec((H0, H1), lambda i: (0, 0)),
            pl.BlockSpec((1, H1), lambda i: (0, 0)),
            pl.BlockSpec((H1, E), lambda i: (0, 0)),
            pl.BlockSpec((1, E), lambda i: (0, 0)),
        ],
        out_specs=pl.BlockSpec((TILE_M, E), lambda i: (i, 0)),
        out_shape=jax.ShapeDtypeStruct((M, E), jnp.float32),
        compiler_params=pltpu.CompilerParams(
            dimension_semantics=("parallel",)),
    )(xf, W0, b0.reshape(1, H0), W1, b1.reshape(1, H1), W2, b2.reshape(1, E))

    logits = logits.reshape(B, S, E)
    return pl.pallas_call(
        _softmax_body,
        grid=(B,),
        in_specs=[pl.BlockSpec((1, S, E), lambda b: (b, 0, 0))],
        out_specs=pl.BlockSpec((1, S, E), lambda b: (b, 0, 0)),
        out_shape=jax.ShapeDtypeStruct((B, S, E), jnp.float32),
    )(logits)


# speedup vs baseline: 1.0471x; 1.0443x over previous
import jax
import jax.numpy as jnp
from jax.experimental import pallas as pl
from jax.experimental.pallas import tpu as pltpu

TILE_S = 1024


def _router_body(x_ref, w0_ref, b0_ref, w1_ref, b1_ref, w2_ref, out_ref,
                 z_ref):
    n_s = pl.num_programs(1)
    s = pl.program_id(1)
    h = jnp.dot(x_ref[0], w0_ref[...], preferred_element_type=jnp.float32)
    h = jnp.maximum(h + b0_ref[...], 0.0)
    h = jnp.dot(h, w1_ref[...], preferred_element_type=jnp.float32)
    h = jnp.maximum(h + b1_ref[...], 0.0)
    z_ref[pl.ds(s * TILE_S, TILE_S), :] = jnp.dot(
        h, w2_ref[...], preferred_element_type=jnp.float32)

    @pl.when(s == n_s - 1)
    def _softmax():
        z = z_ref[...]
        m = jnp.max(z, axis=0, keepdims=True)
        e = jnp.exp(z - m)
        out_ref[0] = e / jnp.sum(e, axis=0, keepdims=True)


def kernel(x, W0, b0, W1, b1, W2, b2):
    B, S, D = x.shape
    H0 = W0.shape[1]
    H1 = W1.shape[1]
    E = W2.shape[1]

    return pl.pallas_call(
        _router_body,
        grid=(B, S // TILE_S),
        in_specs=[
            pl.BlockSpec((1, TILE_S, D), lambda b, s: (b, s, 0)),
            pl.BlockSpec((D, H0), lambda b, s: (0, 0)),
            pl.BlockSpec((1, H0), lambda b, s: (0, 0)),
            pl.BlockSpec((H0, H1), lambda b, s: (0, 0)),
            pl.BlockSpec((1, H1), lambda b, s: (0, 0)),
            pl.BlockSpec((H1, E), lambda b, s: (0, 0)),
        ],
        out_specs=pl.BlockSpec((1, S, E), lambda b, s: (b, 0, 0)),
        out_shape=jax.ShapeDtypeStruct((B, S, E), jnp.float32),
        scratch_shapes=[pltpu.VMEM((S, E), jnp.float32)],
    )(x, W0, b0.reshape(1, H0), W1, b1.reshape(1, H1), W2)
